# baseline (speedup 1.0000x reference)
import functools

import jax
import jax.numpy as jnp
from jax import lax
from jax.experimental import pallas as pl
from jax.experimental.pallas import tpu as pltpu
from jax.experimental.pallas import tpu_sc as plsc

KNN = 16
LANES = 16
NUM_WORKERS = 32
GRP_PTS = 4096
CHUNK_PTS = 32
N_CHUNK_LANES = 128


def _merge_two_sorted(a, b):
  av, ai = a
  bv, bi = b
  bv = lax.rev(bv, (0,))
  bi = lax.rev(bi, (0,))
  keep = av <= bv
  lo_v = jnp.where(keep, av, bv)
  lo_i = jnp.where(keep, ai, bi)
  out_v, out_i = plsc.sort_key_val(lo_v, lo_i)
  return out_v, out_i


def _tree_select16(groups):
  while len(groups) > 1:
    groups = [_merge_two_sorted(groups[k], groups[k + 1])
              for k in range(0, len(groups), 2)]
  return groups[0]


@functools.lru_cache(maxsize=None)
def _make_tc_chunkmin(b_sz, n_pts, k_cen):
  n_vregs = n_pts // 1024
  k_blk = 64
  assert k_cen % k_blk == 0

  def body(x_ref, y_ref, z_ref, cx_ref, cy_ref, cz_ref, o_ref):
    @pl.loop(0, k_blk // 8)
    def kblk(kb):
      sx, sy, sz, m = [], [], [], [None] * 8
      for kk in range(8):
        k = kb * 8 + kk
        sx.append(jnp.full((8, 128), cx_ref[0, 0, 0, k], jnp.float32))
        sy.append(jnp.full((8, 128), cy_ref[0, 0, 0, k], jnp.float32))
        sz.append(jnp.full((8, 128), cz_ref[0, 0, 0, k], jnp.float32))
      for i in range(n_vregs):
        xv = x_ref[0, i]
        yv = y_ref[0, i]
        zv = z_ref[0, i]
        for kk in range(8):
          dx = xv - sx[kk]
          dy = yv - sy[kk]
          dz = zv - sz[kk]
          d2 = dx * dx + dy * dy + dz * dz
          m[kk] = d2 if m[kk] is None else jnp.minimum(m[kk], d2)
      for kk in range(8):
        o_ref[0, kb * 8 + kk] = jnp.minimum(m[kk][0:4], m[kk][4:8])

  pt_spec = pl.BlockSpec((1, n_vregs, 8, 128), lambda b, j: (b, 0, 0, 0))
  cen_spec = pl.BlockSpec((1, 1, 1, k_blk), lambda b, j: (b, j, 0, 0),
                          memory_space=pltpu.SMEM)
  return pl.pallas_call(
      body,
      grid=(b_sz, k_cen // k_blk),
      in_specs=[pt_spec, pt_spec, pt_spec, cen_spec, cen_spec, cen_spec],
      out_specs=pl.BlockSpec((1, k_blk, 4, N_CHUNK_LANES),
                             lambda b, j: (b, j, 0, 0)),
      out_shape=jax.ShapeDtypeStruct(
          (b_sz, k_cen, 4, N_CHUNK_LANES), jnp.float32),
  )


@functools.lru_cache(maxsize=None)
def _make_sc_select(b_sz, n_pts, k_cen):
  assert NUM_WORKERS % b_sz == 0
  workers_per_batch = NUM_WORKERS // b_sz
  cols_per_worker = k_cen // workers_per_batch
  assert cols_per_worker * workers_per_batch == k_cen
  n_grp = n_pts // GRP_PTS
  n_chunks = n_grp * N_CHUNK_LANES

  mesh = plsc.VectorSubcoreMesh(core_axis_name="c", subcore_axis_name="s")

  @functools.partial(
      pl.kernel,
      out_type=jax.ShapeDtypeStruct((b_sz * k_cen * KNN,), jnp.int32),
      mesh=mesh,
      scratch_types=[
          pltpu.VMEM((n_pts,), jnp.float32),
          pltpu.VMEM((n_pts,), jnp.float32),
          pltpu.VMEM((n_pts,), jnp.float32),
          pltpu.VMEM((cols_per_worker,), jnp.float32),
          pltpu.VMEM((cols_per_worker,), jnp.float32),
          pltpu.VMEM((cols_per_worker,), jnp.float32),
          pltpu.VMEM((cols_per_worker * n_chunks,), jnp.float32),
          pltpu.VMEM((cols_per_worker * KNN,), jnp.int32),
      ],
      compiler_params=pltpu.CompilerParams(needs_layout_passes=False),
  )
  def knn(x_h, y_h, z_h, cx_h, cy_h, cz_h, cm_h, out_h,
          x_v, y_v, z_v, cx_v, cy_v, cz_v, cm_v, out_v):
    wid = lax.axis_index("s") * 2 + lax.axis_index("c")
    b = wid // workers_per_batch
    c0 = (wid % workers_per_batch) * cols_per_worker

    pltpu.sync_copy(x_h.at[pl.ds(b * n_pts, n_pts)], x_v)
    pltpu.sync_copy(y_h.at[pl.ds(b * n_pts, n_pts)], y_v)
    pltpu.sync_copy(z_h.at[pl.ds(b * n_pts, n_pts)], z_v)
    pltpu.sync_copy(cx_h.at[pl.ds(b * k_cen + c0, cols_per_worker)], cx_v)
    pltpu.sync_copy(cy_h.at[pl.ds(b * k_cen + c0, cols_per_worker)], cy_v)
    pltpu.sync_copy(cz_h.at[pl.ds(b * k_cen + c0, cols_per_worker)], cz_v)
    pltpu.sync_copy(
        cm_h.at[pl.ds((b * k_cen + c0) * n_chunks,
                      cols_per_worker * n_chunks)], cm_v)

    lane_iota = lax.iota(jnp.int32, LANES)

    @pl.loop(0, cols_per_worker)
    def col_loop(cl):
      cl_v = jnp.full((LANES,), cl, jnp.int32)
      cx = plsc.load_gather(cx_v, [cl_v])
      cy = plsc.load_gather(cy_v, [cl_v])
      cz = plsc.load_gather(cz_v, [cl_v])

      cm_base = cl * n_chunks
      groups = []
      for v in range(n_chunks // LANES):
        vals = cm_v[pl.ds(cm_base + v * LANES, LANES)]
        sv, si = plsc.sort_key_val(vals, lane_iota + v * LANES)
        groups.append((sv, si))
      cand = _tree_select16(groups)[1]

      groups = []
      for i in range(CHUNK_PTS):
        pidx = cand + i * 512
        dx = plsc.load_gather(x_v, [pidx]) - cx
        dy = plsc.load_gather(y_v, [pidx]) - cy
        dz = plsc.load_gather(z_v, [pidx]) - cz
        d2 = dx * dx + dy * dy + dz * dz
        sv, si = plsc.sort_key_val(d2, pidx)
        groups.append((sv, si))
      fin_i = _tree_select16(groups)[1]

      out_v[pl.ds(cl * KNN, KNN)] = fin_i

    pltpu.sync_copy(
        out_v,
        out_h.at[pl.ds((b * k_cen + c0) * KNN, cols_per_worker * KNN)])

  return knn


def _run_half(pts, cen, n_pts, k_cen):
  b_sz = pts.shape[1]
  pts4 = pts.reshape(3, b_sz, n_pts // 1024, 8, 128)
  cenb = cen.reshape(3, b_sz, k_cen // 64, 1, 64)
  tc = _make_tc_chunkmin(b_sz, n_pts, k_cen)
  cm = tc(pts4[0], pts4[1], pts4[2], cenb[0], cenb[1], cenb[2])
  sc = _make_sc_select(b_sz, n_pts, k_cen)
  out = sc(pts[0].reshape(-1), pts[1].reshape(-1), pts[2].reshape(-1),
           cen[0].reshape(-1), cen[1].reshape(-1), cen[2].reshape(-1),
           cm.reshape(-1))
  return jnp.swapaxes(out.reshape(b_sz, k_cen, KNN), 1, 2)


def kernel(xyz, centers):
  b_sz, n_pts, _ = xyz.shape
  k_cen = centers.shape[1]
  pts = jnp.transpose(xyz, (2, 0, 1)).reshape(3, b_sz, n_pts)
  cen = jnp.transpose(centers, (2, 0, 1)).reshape(3, b_sz, k_cen)
  kh = k_cen // 2
  outs = []
  for i in range(b_sz):
    o = [_run_half(pts[:, i:i + 1], cen[:, i:i + 1, j * kh:(j + 1) * kh],
                   n_pts, kh) for j in range(2)]
    outs.append(jnp.concatenate(o, axis=2))
  return jnp.concatenate(outs, axis=0)


# speedup vs baseline: 1.2840x; 1.2840x over previous
import functools

import jax
import jax.numpy as jnp
from jax import lax
from jax.experimental import pallas as pl
from jax.experimental.pallas import tpu as pltpu
from jax.experimental.pallas import tpu_sc as plsc

KNN = 16
LANES = 16
NUM_WORKERS = 32
GRP_PTS = 4096
CHUNK_PTS = 32
N_CHUNK_LANES = 128


def _merge_two_sorted(a, b):
  av, ai = a
  bv, bi = b
  bv = lax.rev(bv, (0,))
  bi = lax.rev(bi, (0,))
  keep = av <= bv
  lo_v = jnp.where(keep, av, bv)
  lo_i = jnp.where(keep, ai, bi)
  out_v, out_i = plsc.sort_key_val(lo_v, lo_i)
  return out_v, out_i


def _tree_select16(groups):
  while len(groups) > 1:
    groups = [_merge_two_sorted(groups[k], groups[k + 1])
              for k in range(0, len(groups), 2)]
  return groups[0]


@functools.lru_cache(maxsize=None)
def _make_tc_chunkmin(b_sz, n_pts, k_cen):
  n_vregs = n_pts // 1024
  k_blk = 64
  assert k_cen % k_blk == 0

  def body(x_ref, y_ref, z_ref, cx_ref, cy_ref, cz_ref, o_ref):
    @pl.loop(0, k_blk // 8)
    def kblk(kb):
      sx, sy, sz, m = [], [], [], [None] * 8
      for kk in range(8):
        k = kb * 8 + kk
        sx.append(jnp.full((8, 128), cx_ref[0, 0, 0, k], jnp.float32))
        sy.append(jnp.full((8, 128), cy_ref[0, 0, 0, k], jnp.float32))
        sz.append(jnp.full((8, 128), cz_ref[0, 0, 0, k], jnp.float32))
      for i in range(n_vregs):
        xv = x_ref[0, i]
        yv = y_ref[0, i]
        zv = z_ref[0, i]
        for kk in range(8):
          dx = xv - sx[kk]
          dy = yv - sy[kk]
          dz = zv - sz[kk]
          d2 = dx * dx + dy * dy + dz * dz
          m[kk] = d2 if m[kk] is None else jnp.minimum(m[kk], d2)
      for kk in range(8):
        o_ref[0, kb * 8 + kk] = jnp.minimum(m[kk][0:4], m[kk][4:8])

  pt_spec = pl.BlockSpec((1, n_vregs, 8, 128), lambda b, j: (b, 0, 0, 0))
  cen_spec = pl.BlockSpec((1, 1, 1, k_blk), lambda b, j: (b, j, 0, 0),
                          memory_space=pltpu.SMEM)
  return pl.pallas_call(
      body,
      grid=(b_sz, k_cen // k_blk),
      in_specs=[pt_spec, pt_spec, pt_spec, cen_spec, cen_spec, cen_spec],
      out_specs=pl.BlockSpec((1, k_blk, 4, N_CHUNK_LANES),
                             lambda b, j: (b, j, 0, 0)),
      out_shape=jax.ShapeDtypeStruct(
          (b_sz, k_cen, 4, N_CHUNK_LANES), jnp.float32),
  )


@functools.lru_cache(maxsize=None)
def _make_sc_select(b_sz, n_pts, k_cen):
  assert NUM_WORKERS % b_sz == 0
  workers_per_batch = NUM_WORKERS // b_sz
  cols_per_worker = k_cen // workers_per_batch
  assert cols_per_worker * workers_per_batch == k_cen
  n_grp = n_pts // GRP_PTS
  n_chunks = n_grp * N_CHUNK_LANES

  mesh = plsc.VectorSubcoreMesh(core_axis_name="c", subcore_axis_name="s")

  @functools.partial(
      pl.kernel,
      out_type=jax.ShapeDtypeStruct((b_sz * k_cen * KNN,), jnp.int32),
      mesh=mesh,
      scratch_types=[
          pltpu.VMEM((n_pts,), jnp.float32),
          pltpu.VMEM((n_pts,), jnp.float32),
          pltpu.VMEM((n_pts,), jnp.float32),
          pltpu.VMEM((cols_per_worker,), jnp.float32),
          pltpu.VMEM((cols_per_worker,), jnp.float32),
          pltpu.VMEM((cols_per_worker,), jnp.float32),
          pltpu.VMEM((cols_per_worker * n_chunks,), jnp.float32),
          pltpu.VMEM((cols_per_worker * KNN,), jnp.int32),
      ],
      compiler_params=pltpu.CompilerParams(needs_layout_passes=False),
  )
  def knn(x_h, y_h, z_h, cx_h, cy_h, cz_h, cm_h, out_h,
          x_v, y_v, z_v, cx_v, cy_v, cz_v, cm_v, out_v):
    wid = lax.axis_index("s") * 2 + lax.axis_index("c")
    b = wid // workers_per_batch
    c0 = (wid % workers_per_batch) * cols_per_worker

    pltpu.sync_copy(x_h.at[pl.ds(b * n_pts, n_pts)], x_v)
    pltpu.sync_copy(y_h.at[pl.ds(b * n_pts, n_pts)], y_v)
    pltpu.sync_copy(z_h.at[pl.ds(b * n_pts, n_pts)], z_v)
    pltpu.sync_copy(cx_h.at[pl.ds(b * k_cen + c0, cols_per_worker)], cx_v)
    pltpu.sync_copy(cy_h.at[pl.ds(b * k_cen + c0, cols_per_worker)], cy_v)
    pltpu.sync_copy(cz_h.at[pl.ds(b * k_cen + c0, cols_per_worker)], cz_v)
    pltpu.sync_copy(
        cm_h.at[pl.ds((b * k_cen + c0) * n_chunks,
                      cols_per_worker * n_chunks)], cm_v)

    lane_iota = lax.iota(jnp.int32, LANES)

    @pl.loop(0, cols_per_worker)
    def col_loop(cl):
      cl_v = jnp.full((LANES,), cl, jnp.int32)
      cx = plsc.load_gather(cx_v, [cl_v])
      cy = plsc.load_gather(cy_v, [cl_v])
      cz = plsc.load_gather(cz_v, [cl_v])

      cm_base = cl * n_chunks
      groups = []
      for v in range(n_chunks // LANES):
        vals = cm_v[pl.ds(cm_base + v * LANES, LANES)]
        sv, si = plsc.sort_key_val(vals, lane_iota + v * LANES)
        groups.append((sv, si))
      cand = _tree_select16(groups)[1]

      groups = []
      for i in range(CHUNK_PTS):
        pidx = cand + i * 512
        dx = plsc.load_gather(x_v, [pidx]) - cx
        dy = plsc.load_gather(y_v, [pidx]) - cy
        dz = plsc.load_gather(z_v, [pidx]) - cz
        d2 = dx * dx + dy * dy + dz * dz
        sv, si = plsc.sort_key_val(d2, pidx)
        groups.append((sv, si))
      fin_i = _tree_select16(groups)[1]

      out_v[pl.ds(cl * KNN, KNN)] = fin_i

    pltpu.sync_copy(
        out_v,
        out_h.at[pl.ds((b * k_cen + c0) * KNN, cols_per_worker * KNN)])

  return knn


def _run_half(pts, cen, n_pts, k_cen):
  b_sz = pts.shape[1]
  pts4 = pts.reshape(3, b_sz, n_pts // 1024, 8, 128)
  cenb = cen.reshape(3, b_sz, k_cen // 64, 1, 64)
  tc = _make_tc_chunkmin(b_sz, n_pts, k_cen)
  cm = tc(pts4[0], pts4[1], pts4[2], cenb[0], cenb[1], cenb[2])
  sc = _make_sc_select(b_sz, n_pts, k_cen)
  out = sc(pts[0].reshape(-1), pts[1].reshape(-1), pts[2].reshape(-1),
           cen[0].reshape(-1), cen[1].reshape(-1), cen[2].reshape(-1),
           cm.reshape(-1))
  return jnp.swapaxes(out.reshape(b_sz, k_cen, KNN), 1, 2)


def kernel(xyz, centers):
  b_sz, n_pts, _ = xyz.shape
  k_cen = centers.shape[1]
  pts = jnp.transpose(xyz, (2, 0, 1)).reshape(3, b_sz, n_pts)
  cen = jnp.transpose(centers, (2, 0, 1)).reshape(3, b_sz, k_cen)
  outs = [_run_half(pts[:, i:i + 1], cen[:, i:i + 1], n_pts, k_cen)
          for i in range(b_sz)]
  return jnp.concatenate(outs, axis=0)
